# baseline (speedup 1.0000x reference)
import functools

import jax
import jax.numpy as jnp
from jax import lax
from jax.experimental import pallas as pl
from jax.experimental.pallas import tpu as pltpu
from jax.experimental.pallas import tpu_sc as plsc

_NC, _NS, _L = 2, 16, 16
_NW = _NC * _NS
_CH = 128


def _sqrt16(x):
    i = plsc.bitcast(x, jnp.int32)
    i = (i >> 1) + jnp.int32(0x1FBD1DF5)
    y = plsc.bitcast(i, jnp.float32)
    for _ in range(3):
        y = 0.5 * (y + x / y)
    return y


@functools.partial(jax.jit, static_argnames=())
def _transe_sc(head_idx, tail_idx, entity_emb, relation_emb):
    B = head_idx.shape[0]
    D = entity_emb.shape[1]
    bpw = B // _NW
    nch = bpw // _CH
    ngroups = bpw // _L

    mesh = plsc.VectorSubcoreMesh(core_axis_name="c", subcore_axis_name="s")

    @functools.partial(
        pl.kernel,
        out_type=jax.ShapeDtypeStruct((B,), jnp.float32),
        mesh=mesh,
        compiler_params=pltpu.CompilerParams(needs_layout_passes=False,
                                             use_tc_tiling_on_sc=False),
        scratch_types=[
            pltpu.VMEM((bpw,), jnp.int32),
            pltpu.VMEM((bpw,), jnp.int32),
            pltpu.VMEM((bpw, D), jnp.float32),
            pltpu.VMEM((bpw, D), jnp.float32),
            pltpu.VMEM((D,), jnp.float32),
            pltpu.VMEM((bpw,), jnp.float32),
            pltpu.SemaphoreType.DMA,
        ],
    )
    def k(hidx_hbm, tidx_hbm, table_hbm, rel_hbm, out_hbm,
          hidx_v, tidx_v, hrows_v, trows_v, rel_v, out_v, sem):
        wid = lax.axis_index("s") * _NC + lax.axis_index("c")
        base = wid * bpw
        pltpu.sync_copy(hidx_hbm.at[pl.ds(base, bpw)], hidx_v)
        pltpu.sync_copy(tidx_hbm.at[pl.ds(base, bpw)], tidx_v)
        pltpu.sync_copy(rel_hbm.at[0], rel_v)

        copies = []
        for c in range(nch):
            sl = pl.ds(c * _CH, _CH)
            copies.append(pltpu.async_copy(table_hbm.at[hidx_v.at[sl]],
                                           hrows_v.at[sl], sem))
            copies.append(pltpu.async_copy(table_hbm.at[tidx_v.at[sl]],
                                           trows_v.at[sl], sem))
        for cp in copies:
            cp.wait()

        lane = lax.iota(jnp.int32, _L)

        def group_body(g, _):
            row0 = g * _L
            ridx = row0 + lane

            def dim_body(d, acc):
                didx = jnp.full((_L,), d, jnp.int32)
                h = plsc.load_gather(hrows_v, [ridx, didx])
                t = plsc.load_gather(trows_v, [ridx, didx])
                r = plsc.load_gather(rel_v, [didx])
                diff = h - t + r
                return acc + diff * diff

            acc = lax.fori_loop(0, D, dim_body, jnp.zeros((_L,), jnp.float32))
            s = _sqrt16(acc)
            out_v[pl.ds(row0, _L)] = 1.0 / (1.0 + jnp.exp(s))
            return 0

        lax.fori_loop(0, ngroups, group_body, 0)
        pltpu.sync_copy(out_v, out_hbm.at[pl.ds(base, bpw)])

    return k(head_idx, tail_idx, entity_emb, relation_emb)


def kernel(edge_index, entity_emb, relation_emb):
    return _transe_sc(edge_index[0], edge_index[1], entity_emb, relation_emb)


# speedup vs baseline: 1.1000x; 1.1000x over previous
import functools

import jax
import jax.numpy as jnp
from jax import lax
from jax.experimental import pallas as pl
from jax.experimental.pallas import tpu as pltpu
from jax.experimental.pallas import tpu_sc as plsc

_NC, _NS, _L = 2, 16, 16
_NW = _NC * _NS
_CH = 128


def _sqrt16(x):
    i = plsc.bitcast(x, jnp.int32)
    i = (i >> 1) + jnp.int32(0x1FBD1DF5)
    y = plsc.bitcast(i, jnp.float32)
    for _ in range(3):
        y = 0.5 * (y + x / y)
    return y


def _transe_sc(head_idx, tail_idx, table2, rel_f32):
    B = head_idx.shape[0]
    W = table2.shape[1]
    D = W // 2
    bpw = B // _NW
    nch = bpw // _CH
    gpc = _CH // _L

    mesh = plsc.VectorSubcoreMesh(core_axis_name="c", subcore_axis_name="s")

    @functools.partial(
        pl.kernel,
        out_type=jax.ShapeDtypeStruct((B,), jnp.float32),
        mesh=mesh,
        compiler_params=pltpu.CompilerParams(needs_layout_passes=False,
                                             use_tc_tiling_on_sc=True),
        scratch_types=[
            pltpu.VMEM((bpw,), jnp.int32),
            pltpu.VMEM((bpw,), jnp.int32),
            pltpu.VMEM((2, _CH, W), jnp.float32),
            pltpu.VMEM((2, _CH, W), jnp.float32),
            pltpu.VMEM((D,), jnp.float32),
            pltpu.VMEM((bpw,), jnp.float32),
            pltpu.SemaphoreType.DMA,
        ],
    )
    def k(hidx_hbm, tidx_hbm, table_hbm, rel_hbm, out_hbm,
          hidx_v, tidx_v, hbuf, tbuf, rel_v, out_v, sem):
        wid = lax.axis_index("s") * _NC + lax.axis_index("c")
        base = wid * bpw
        pltpu.sync_copy(hidx_hbm.at[pl.ds(base, bpw)], hidx_v)
        pltpu.sync_copy(tidx_hbm.at[pl.ds(base, bpw)], tidx_v)
        pltpu.sync_copy(rel_hbm, rel_v)

        def fire(c):
            sl = pl.ds(c * _CH, _CH)
            cp_h = pltpu.async_copy(table_hbm.at[hidx_v.at[sl]],
                                    hbuf.at[c % 2], sem)
            cp_t = pltpu.async_copy(table_hbm.at[tidx_v.at[sl]],
                                    tbuf.at[c % 2], sem)
            return (cp_h, cp_t)

        lane = lax.iota(jnp.int32, _L)
        inflight = [fire(0), fire(1)]

        for c in range(nch):
            for cp in inflight.pop(0):
                cp.wait()
            bidx = jnp.full((_L,), c % 2, jnp.int32)

            def group_body(g, _, c=c, bidx=bidx):
                row0 = c * _CH + g * _L
                ridx = lane + g * _L
                acc = jnp.zeros((_L,), jnp.float32)
                for d in range(D):
                    didx = jnp.full((_L,), d, jnp.int32)
                    h = plsc.load_gather(hbuf, [bidx, ridx, didx])
                    t = plsc.load_gather(tbuf, [bidx, ridx, didx])
                    r = plsc.load_gather(rel_v, [didx])
                    diff = h - t + r
                    acc = acc + diff * diff
                s = _sqrt16(acc)
                out_v[pl.ds(row0, _L)] = 1.0 / (1.0 + jnp.exp(s))
                return 0

            lax.fori_loop(0, gpc, group_body, 0)
            if c + 2 < nch:
                inflight.append(fire(c + 2))

        pltpu.sync_copy(out_v, out_hbm.at[pl.ds(base, bpw)])

    return k(head_idx, tail_idx, table2, rel_f32)


def kernel(edge_index, entity_emb, relation_emb):
    n, d = entity_emb.shape
    table2 = jnp.pad(entity_emb, ((0, 0), (0, d)))
    return _transe_sc(edge_index[0], edge_index[1], table2,
                      relation_emb.reshape(-1))


# speedup vs baseline: 1.1117x; 1.0106x over previous
import functools

import jax
import jax.numpy as jnp
from jax import lax
from jax.experimental import pallas as pl
from jax.experimental.pallas import tpu as pltpu
from jax.experimental.pallas import tpu_sc as plsc

_NC, _NS, _L = 2, 16, 16
_NW = _NC * _NS
_CH = 128


def _sqrt16(x):
    i = plsc.bitcast(x, jnp.int32)
    i = (i >> 1) + jnp.int32(0x1FBD1DF5)
    y = plsc.bitcast(i, jnp.float32)
    for _ in range(3):
        y = 0.5 * (y + x / y)
    return y


def _transe_sc(head_idx, tail_idx, table2, rel_f32):
    B = head_idx.shape[0]
    W = table2.shape[1]
    D = W // 2
    bpw = B // _NW
    nch = bpw // _CH
    gpc = _CH // _L

    mesh = plsc.VectorSubcoreMesh(core_axis_name="c", subcore_axis_name="s")

    @functools.partial(
        pl.kernel,
        out_type=jax.ShapeDtypeStruct((B,), jnp.float32),
        mesh=mesh,
        compiler_params=pltpu.CompilerParams(needs_layout_passes=False,
                                             use_tc_tiling_on_sc=True),
        scratch_types=[
            pltpu.VMEM((bpw,), jnp.int32),
            pltpu.VMEM((bpw,), jnp.int32),
            pltpu.VMEM((2, _CH, W), jnp.float32),
            pltpu.VMEM((2, _CH, W), jnp.float32),
            pltpu.VMEM((D,), jnp.float32),
            pltpu.VMEM((bpw,), jnp.float32),
            pltpu.SemaphoreType.DMA,
        ],
    )
    def k(hidx_hbm, tidx_hbm, table_hbm, rel_hbm, out_hbm,
          hidx_v, tidx_v, hbuf, tbuf, rel_v, out_v, sem):
        wid = lax.axis_index("s") * _NC + lax.axis_index("c")
        base = wid * bpw
        pltpu.sync_copy(hidx_hbm.at[pl.ds(base, bpw)], hidx_v)
        pltpu.sync_copy(tidx_hbm.at[pl.ds(base, bpw)], tidx_v)
        pltpu.sync_copy(rel_hbm, rel_v)

        def fire(c):
            sl = pl.ds(c * _CH, _CH)
            cp_h = pltpu.async_copy(table_hbm.at[hidx_v.at[sl]],
                                    hbuf.at[c % 2], sem)
            cp_t = pltpu.async_copy(table_hbm.at[tidx_v.at[sl]],
                                    tbuf.at[c % 2], sem)
            return (cp_h, cp_t)

        lane = lax.iota(jnp.int32, _L)
        inflight = [fire(0), fire(1)]

        for c in range(nch):
            for cp in inflight.pop(0):
                cp.wait()
            bidx = jnp.full((_L,), c % 2, jnp.int32)

            def group_body(g, _, c=c, bidx=bidx):
                row0 = c * _CH + g * _L
                ridx = lane + g * _L
                def dim_body(d, acc):
                    didx = jnp.full((_L,), d, jnp.int32)
                    h = plsc.load_gather(hbuf, [bidx, ridx, didx])
                    t = plsc.load_gather(tbuf, [bidx, ridx, didx])
                    r = plsc.load_gather(rel_v, [didx])
                    diff = h - t + r
                    return acc + diff * diff

                acc = lax.fori_loop(0, D, dim_body,
                                    jnp.zeros((_L,), jnp.float32))
                s = _sqrt16(acc)
                out_v[pl.ds(row0, _L)] = 1.0 / (1.0 + jnp.exp(s))
                return 0

            lax.fori_loop(0, gpc, group_body, 0)
            if c + 2 < nch:
                inflight.append(fire(c + 2))

        pltpu.sync_copy(out_v, out_hbm.at[pl.ds(base, bpw)])

    return k(head_idx, tail_idx, table2, rel_f32)


def kernel(edge_index, entity_emb, relation_emb):
    n, d = entity_emb.shape
    table2 = jnp.pad(entity_emb, ((0, 0), (0, d)))
    return _transe_sc(edge_index[0], edge_index[1], table2,
                      relation_emb.reshape(-1))


# speedup vs baseline: 1.6532x; 1.4872x over previous
import functools

import jax
import jax.numpy as jnp
from jax import lax
from jax.experimental import pallas as pl
from jax.experimental.pallas import tpu as pltpu
from jax.experimental.pallas import tpu_sc as plsc

_NC, _NS, _L = 2, 16, 16
_NW = _NC * _NS
_CH = 128


def _sqrt16(x):
    i = plsc.bitcast(x, jnp.int32)
    i = (i >> 1) + jnp.int32(0x1FBD1DF5)
    y = plsc.bitcast(i, jnp.float32)
    for _ in range(3):
        y = 0.5 * (y + x / y)
    return y


def _transe_sc(head_idx, tail_idx, table, rel_f32):
    B = head_idx.shape[0]
    D = table.shape[1]
    bpw = B // _NW
    nch = bpw // _CH
    gpc = _CH // _L

    mesh = plsc.VectorSubcoreMesh(core_axis_name="c", subcore_axis_name="s")

    @functools.partial(
        pl.kernel,
        out_type=jax.ShapeDtypeStruct((B,), jnp.float32),
        mesh=mesh,
        compiler_params=pltpu.CompilerParams(needs_layout_passes=False,
                                             use_tc_tiling_on_sc=True),
        scratch_types=[
            pltpu.VMEM((bpw,), jnp.int32),
            pltpu.VMEM((bpw,), jnp.int32),
            pltpu.VMEM((2, _CH, D), jnp.float32),
            pltpu.VMEM((2, _CH, D), jnp.float32),
            pltpu.VMEM((D,), jnp.float32),
            pltpu.VMEM((bpw,), jnp.float32),
            pltpu.SemaphoreType.DMA,
        ],
    )
    def k(hidx_hbm, tidx_hbm, table_hbm, rel_hbm, out_hbm,
          hidx_v, tidx_v, hbuf, tbuf, rel_v, out_v, sem):
        wid = lax.axis_index("s") * _NC + lax.axis_index("c")
        base = wid * bpw
        pltpu.sync_copy(hidx_hbm.at[pl.ds(base, bpw)], hidx_v)
        pltpu.sync_copy(tidx_hbm.at[pl.ds(base, bpw)], tidx_v)
        pltpu.sync_copy(rel_hbm, rel_v)

        def fire(c):
            buf = c % 2

            def fire_group(g, _):
                hv = hidx_v[pl.ds(c * _CH + g * _L, _L)]
                tv = tidx_v[pl.ds(c * _CH + g * _L, _L)]
                for l in range(_L):
                    row = g * _L + l
                    pltpu.async_copy(table_hbm.at[hv[l]],
                                     hbuf.at[buf, row], sem)
                    pltpu.async_copy(table_hbm.at[tv[l]],
                                     tbuf.at[buf, row], sem)
                return 0

            lax.fori_loop(0, gpc, fire_group, 0)

        def drain(c):
            buf = c % 2
            pltpu.make_async_copy(table_hbm.at[pl.ds(0, _CH)],
                                  hbuf.at[buf], sem).wait()
            pltpu.make_async_copy(table_hbm.at[pl.ds(0, _CH)],
                                  tbuf.at[buf], sem).wait()

        lane = lax.iota(jnp.int32, _L)
        fire(0)
        fire(1)

        for c in range(nch):
            drain(c)
            bidx = jnp.full((_L,), c % 2, jnp.int32)

            def group_body(g, _, c=c, bidx=bidx):
                row0 = c * _CH + g * _L
                ridx = lane + g * _L

                def dim_body(d, acc):
                    didx = jnp.full((_L,), d, jnp.int32)
                    h = plsc.load_gather(hbuf, [bidx, ridx, didx])
                    t = plsc.load_gather(tbuf, [bidx, ridx, didx])
                    r = plsc.load_gather(rel_v, [didx])
                    diff = h - t + r
                    return acc + diff * diff

                acc = lax.fori_loop(0, D, dim_body,
                                    jnp.zeros((_L,), jnp.float32))
                s = _sqrt16(acc)
                out_v[pl.ds(row0, _L)] = 1.0 / (1.0 + jnp.exp(s))
                return 0

            lax.fori_loop(0, gpc, group_body, 0)
            if c + 2 < nch:
                fire(c + 2)

        pltpu.sync_copy(out_v, out_hbm.at[pl.ds(base, bpw)])

    return k(head_idx, tail_idx, table, rel_f32)


def kernel(edge_index, entity_emb, relation_emb):
    return _transe_sc(edge_index[0], edge_index[1], entity_emb,
                      relation_emb.reshape(-1))
